# baseline (speedup 1.0000x reference)
import jax
import jax.numpy as jnp
import numpy as np
from jax.experimental import pallas as pl


def _conv(x, src, dst, wq, bq, wk, bk, wv, bv, ws, bs, heads, out_ch):
    n = x.shape[0]
    q = (x @ wq + bq).reshape(n, heads, out_ch)
    k = (x @ wk + bk).reshape(n, heads, out_ch)
    v = (x @ wv + bv).reshape(n, heads, out_ch)
    alpha = jnp.sum(q[dst] * k[src], axis=-1) / np.sqrt(out_ch)
    amax = jax.ops.segment_max(alpha, dst, num_segments=n)
    amax = jnp.where(jnp.isfinite(amax), amax, 0.0)
    ex = jnp.exp(alpha - amax[dst])
    denom = jax.ops.segment_sum(ex, dst, num_segments=n)
    attn = ex / (denom[dst] + 1e-16)
    msg = v[src] * attn[:, :, None]
    out = jax.ops.segment_sum(msg, dst, num_segments=n).reshape(n, heads * out_ch)
    return out + x @ ws + bs


def _copy_kernel(x_ref, o_ref):
    o_ref[...] = x_ref[...]


def kernel(x, edge_index, batch, wq1, bq1, wk1, bk1, wv1, bv1, ws1, bs1, wq2, bq2, wk2, bk2, wv2, bv2, ws2, bs2, w_lin1, b_lin1, w_lin2, b_lin2, w_lin3, b_lin3):
    src = edge_index[0]
    dst = edge_index[1]
    h = jax.nn.relu(_conv(x, src, dst, wq1, bq1, wk1, bk1, wv1, bv1, ws1, bs1, 2, 32))
    h = jax.nn.relu(_conv(h, src, dst, wq2, bq2, wk2, bk2, wv2, bv2, ws2, bs2, 2, 64))
    G = 64
    pooled = jax.ops.segment_max(h, batch, num_segments=G)
    pooled = jnp.where(jnp.isfinite(pooled), pooled, 0.0)
    x_latent = jax.nn.relu(pooled @ w_lin1 + b_lin1)
    h2 = jax.nn.relu(x_latent @ w_lin2 + b_lin2)
    logits = h2 @ w_lin3 + b_lin3
    logits = pl.pallas_call(
        _copy_kernel,
        out_shape=jax.ShapeDtypeStruct(logits.shape, logits.dtype),
    )(logits)
    return (logits, x_latent)


# speedup vs baseline: 7.9400x; 7.9400x over previous
import functools
import jax
import jax.numpy as jnp
import numpy as np
from jax import lax
from jax.experimental import pallas as pl
from jax.experimental.pallas import tpu as pltpu
from jax.experimental.pallas import tpu_sc as plsc

N = 50000
E = 800000
G = 64
NSC = 2
TPS = 16
NW = NSC * TPS
EPT = 25088
E_PAD = EPT * NW
N_PAD = 50176
NEG = -1e30


def _mm_body(x_ref, w_ref, b_ref, o_ref):
    o_ref[...] = (
        jnp.dot(x_ref[...], w_ref[...], preferred_element_type=jnp.float32)
        + b_ref[...]
    )


def _matmul(x, w, b, blk=1000):
    n, din = x.shape
    dout = w.shape[1]
    return pl.pallas_call(
        _mm_body,
        grid=(n // blk,),
        in_specs=[
            pl.BlockSpec((blk, din), lambda i: (i, 0)),
            pl.BlockSpec((din, dout), lambda i: (0, 0)),
            pl.BlockSpec((1, dout), lambda i: (0, 0)),
        ],
        out_specs=pl.BlockSpec((blk, dout), lambda i: (i, 0)),
        out_shape=jax.ShapeDtypeStruct((n, dout), jnp.float32),
    )(x, w, b.reshape(1, dout))


def _merge_max_body(t_ref, o_ref):
    m = jnp.max(t_ref[...], axis=0)
    o_ref[...] = jnp.where(m < -1e29, 0.0, m)


def _merge_max(tabs, blk=3584):
    nt, two, m = tabs.shape
    return pl.pallas_call(
        _merge_max_body,
        grid=(m // blk,),
        in_specs=[pl.BlockSpec((nt, two, blk), lambda i: (0, 0, i))],
        out_specs=pl.BlockSpec((two, blk), lambda i: (0, i)),
        out_shape=jax.ShapeDtypeStruct((two, m), jnp.float32),
    )(tabs)


def _merge_sum_body(t_ref, o_ref):
    o_ref[...] = jnp.sum(t_ref[...], axis=0)


def _merge_sum(tabs, blk=3584):
    nt, two, m = tabs.shape
    return pl.pallas_call(
        _merge_sum_body,
        grid=(m // blk,),
        in_specs=[pl.BlockSpec((nt, two, blk), lambda i: (0, 0, i))],
        out_specs=pl.BlockSpec((two, blk), lambda i: (0, i)),
        out_shape=jax.ShapeDtypeStruct((two, m), jnp.float32),
    )(tabs)


def _norm_body(acc_ref, dn_ref, skip_ref, o_ref, *, C):
    acc = acc_ref[0] + acc_ref[1]
    dn = dn_ref[...]
    blk = acc.shape[0]
    dnb = jnp.concatenate(
        [
            jnp.broadcast_to(dn[:, 0:1], (blk, C)),
            jnp.broadcast_to(dn[:, 1:2], (blk, C)),
        ],
        axis=1,
    )
    h = acc[:, :2 * C] / (dnb + 1e-16) + skip_ref[...]
    o_ref[...] = jnp.maximum(h, 0.0)


def _norm(acc_parts, denom2d, skip, C, blk=1000):
    n, d = skip.shape
    d2 = acc_parts.shape[2]
    return pl.pallas_call(
        functools.partial(_norm_body, C=C),
        grid=(n // blk,),
        in_specs=[
            pl.BlockSpec((2, blk, d2), lambda i: (0, i, 0)),
            pl.BlockSpec((blk, 8), lambda i: (i, 0)),
            pl.BlockSpec((blk, d), lambda i: (i, 0)),
        ],
        out_specs=pl.BlockSpec((blk, d), lambda i: (i, 0)),
        out_shape=jax.ShapeDtypeStruct((n, d), jnp.float32),
    )(acc_parts, denom2d, skip)


def _mlp_body(tabs_ref, w1_ref, b1_ref, w2_ref, b2_ref, w3_ref, b3_ref,
              logits_ref, xlat_ref):
    pooled = jnp.max(tabs_ref[...], axis=0)
    xl = jnp.maximum(
        jnp.dot(pooled, w1_ref[...], preferred_element_type=jnp.float32)
        + b1_ref[...], 0.0)
    h = jnp.maximum(
        jnp.dot(xl, w2_ref[...], preferred_element_type=jnp.float32)
        + b2_ref[...], 0.0)
    logits_ref[...] = (
        jnp.dot(h, w3_ref[...], preferred_element_type=jnp.float32)
        + b3_ref[...])
    xlat_ref[...] = xl


def _mlp(pool_tabs, w1, b1, w2, b2, w3, b3):
    nc = w3.shape[1]
    return pl.pallas_call(
        _mlp_body,
        out_shape=[
            jax.ShapeDtypeStruct((G, nc), jnp.float32),
            jax.ShapeDtypeStruct((G, 32), jnp.float32),
        ],
    )(pool_tabs, w1, b1.reshape(1, 32), w2, b2.reshape(1, 128),
      w3, b3.reshape(1, nc))


def _make_conv_a(D, C, CH, DV=128):
    ITERS = EPT // CH
    mesh = plsc.VectorSubcoreMesh(core_axis_name="c", subcore_axis_name="s",
                                  num_cores=NSC, num_subcores=TPS)
    inv_sqrt_c = float(1.0 / np.sqrt(C))

    @functools.partial(
        pl.kernel,
        out_type=[
            jax.ShapeDtypeStruct((2 * E_PAD,), jnp.float32),
            jax.ShapeDtypeStruct((NW, 2, N_PAD), jnp.float32),
        ],
        mesh=mesh,
        compiler_params=pltpu.CompilerParams(needs_layout_passes=False),
        scratch_types=[
            pltpu.VMEM((CH + 16,), jnp.int32),
            pltpu.VMEM((CH,), jnp.int32),
            pltpu.VMEM((CH, DV), jnp.float32),
            pltpu.VMEM((CH, DV), jnp.float32),
            pltpu.VMEM((2, CH + 16), jnp.float32),
            pltpu.VMEM((N_PAD,), jnp.float32),
            pltpu.VMEM((N_PAD,), jnp.float32),
            pltpu.SemaphoreType.DMA,
        ],
    )
    def conv_a(q_hbm, k_hbm, src_hbm, dst_hbm, alpha_out, tabs_out,
               dstbuf, srcbuf, qrows, krows, alphabuf, tab0, tab1, sem):
        cid = lax.axis_index("c")
        sid = lax.axis_index("s")
        wid = sid * NSC + cid
        iota = lax.iota(jnp.int32, 16)
        lane0 = iota == 0
        negv = jnp.full((16,), NEG, jnp.float32)

        def init_body(i, carry):
            tab0[pl.ds(i * 16, 16)] = negv
            tab1[pl.ds(i * 16, 16)] = negv
            return carry

        lax.fori_loop(0, N_PAD // 16, init_body, 0)

        base0 = wid * EPT

        def chunk_body(it, carry):
            base = base0 + it * CH
            pltpu.sync_copy(dst_hbm.at[pl.ds(base, CH)],
                            dstbuf.at[pl.ds(0, CH)])
            pltpu.sync_copy(src_hbm.at[pl.ds(base, CH)], srcbuf)
            pltpu.async_copy(q_hbm.at[dstbuf.at[pl.ds(0, CH)]], qrows,
                             sem).wait()
            pltpu.async_copy(k_hbm.at[srcbuf], krows, sem).wait()

            for j in range(CH // 16):
                sl = pl.ds(j * 16, 16)
                e16 = iota + j * 16
                for h in range(2):
                    acc = jnp.zeros((16,), jnp.float32)
                    for cc in range(C):
                        col = jnp.full((16,), h * C + cc, jnp.int32)
                        qc = plsc.load_gather(qrows, [e16, col])
                        kc = plsc.load_gather(krows, [e16, col])
                        acc = acc + qc * kc
                    alphabuf[h, sl] = acc * inv_sqrt_c

            def edge_body(e, ecarry):
                d = dstbuf[pl.ds(e, 16)][0]
                valid = (base + e) < E
                for h, tab in ((0, tab0), (1, tab1)):
                    a = alphabuf[h, pl.ds(e, 16)][0]
                    a_eff = jnp.where(valid, a, NEG)
                    cur = tab[pl.ds(d, 16)][0]
                    new = jnp.maximum(cur, a_eff)
                    plsc.store_scatter(tab, [jnp.full((16,), d, jnp.int32)],
                                       jnp.full((16,), new, jnp.float32),
                                       mask=lane0)
                return ecarry

            lax.fori_loop(0, CH, edge_body, 0)
            pltpu.sync_copy(alphabuf.at[0, pl.ds(0, CH)],
                            alpha_out.at[pl.ds(base, CH)])
            pltpu.sync_copy(alphabuf.at[1, pl.ds(0, CH)],
                            alpha_out.at[pl.ds(E_PAD + base, CH)])
            return carry

        lax.fori_loop(0, ITERS, chunk_body, 0)
        pltpu.sync_copy(tab0, tabs_out.at[wid, 0])
        pltpu.sync_copy(tab1, tabs_out.at[wid, 1])

    return conv_a


def _make_conv_c(D, C, CH, CH2, R, RNG, EMAX, DV=128):
    ITERS = EPT // CH
    RPT = RNG // TPS
    ZCH = 8
    ZITERS = RPT // ZCH
    NJ = CH // 16
    D2 = 128
    mesh = plsc.VectorSubcoreMesh(core_axis_name="c", subcore_axis_name="s",
                                  num_cores=NSC, num_subcores=TPS)

    @functools.partial(
        pl.kernel,
        out_type=[
            jax.ShapeDtypeStruct((NSC, N_PAD, D2), jnp.float32),
            jax.ShapeDtypeStruct((NW, 2, N_PAD), jnp.float32),
        ],
        mesh=mesh,
        compiler_params=pltpu.CompilerParams(needs_layout_passes=False),
        scratch_types=[
            pltpu.VMEM((CH,), jnp.int32),
            pltpu.VMEM((CH,), jnp.int32),
            pltpu.VMEM((2, CH), jnp.float32),
            pltpu.VMEM((EMAX + 16,), jnp.int32),
            pltpu.VMEM((EMAX + 16,), jnp.int32),
            pltpu.VMEM((EMAX + 16,), jnp.float32),
            pltpu.VMEM((EMAX + 16,), jnp.float32),
            pltpu.VMEM((CH2, DV), jnp.float32),
            pltpu.VMEM((CH2, D2), jnp.float32),
            pltpu.VMEM((RNG,), jnp.float32),
            pltpu.VMEM((RNG,), jnp.float32),
            pltpu.VMEM((ZCH, D2), jnp.float32),
            pltpu.VMEM((RNG,), jnp.float32),
            pltpu.VMEM((RNG,), jnp.float32),
            pltpu.VMEM_SHARED((RNG, D2), jnp.float32),
            pltpu.SemaphoreType.DMA,
        ],
    )
    def conv_c(v_hbm, src_hbm, dst_hbm, alpha_hbm, amax_hbm,
               acc_out, dtab_out,
               dstbuf, srcbuf, alphabuf, lidx_c, src_c, ex0_c, ex1_c,
               vrows, msgbuf, amax_q0, amax_q1, zbuf, dtab0, dtab1,
               acc_sh, sem):
        cid = lax.axis_index("c")
        sid = lax.axis_index("s")
        wid = sid * NSC + cid
        base0 = wid * EPT

        zv = jnp.zeros((16,), jnp.float32)
        ziv = jnp.zeros((16,), jnp.int32)
        iota = lax.iota(jnp.int32, 16)
        lane0 = iota == 0

        def zb_body(i, carry):
            for jj in range(D2 // 16):
                zbuf[i, pl.ds(jj * 16, 16)] = zv
            return carry

        lax.fori_loop(0, ZCH, zb_body, 0)

        def mz_body(i, carry):
            for jj in range(D2 // 16):
                msgbuf[i, pl.ds(jj * 16, 16)] = zv
            return carry

        lax.fori_loop(0, CH2, mz_body, 0)

        def range_body(r, carry):
            rbase = r * RNG
            pltpu.sync_copy(amax_hbm.at[pl.ds(rbase, RNG)], amax_q0)
            pltpu.sync_copy(amax_hbm.at[pl.ds(N_PAD + rbase, RNG)], amax_q1)

            def cz_body(i, c2):
                sl = pl.ds(i * 16, 16)
                lidx_c[sl] = ziv
                src_c[sl] = ziv
                ex0_c[sl] = zv
                ex1_c[sl] = zv
                return c2

            lax.fori_loop(0, (EMAX + 16) // 16, cz_body, 0)

            def dz_body(i, c2):
                sl = pl.ds(i * 16, 16)
                dtab0[sl] = zv
                dtab1[sl] = zv
                return c2

            lax.fori_loop(0, RNG // 16, dz_body, 0)

            def zero_sh(z, c2):
                pltpu.sync_copy(zbuf,
                                acc_sh.at[pl.ds(sid * RPT + z * ZCH, ZCH)])
                return c2

            lax.fori_loop(0, ZITERS, zero_sh, 0)
            plsc.subcore_barrier()

            def chunk_scan(it, cur):
                base = base0 + it * CH
                pltpu.sync_copy(dst_hbm.at[pl.ds(base, CH)], dstbuf)
                pltpu.sync_copy(src_hbm.at[pl.ds(base, CH)], srcbuf)
                pltpu.sync_copy(alpha_hbm.at[pl.ds(base, CH)],
                                alphabuf.at[0])
                pltpu.sync_copy(alpha_hbm.at[pl.ds(E_PAD + base, CH)],
                                alphabuf.at[1])
                for j in range(NJ):
                    sl = pl.ds(j * 16, 16)
                    dst16 = dstbuf[sl]
                    src16 = srcbuf[sl]
                    eid = iota + (base + j * 16)
                    inr = ((dst16 >= rbase) & (dst16 < rbase + RNG)
                           & (eid < E))
                    lidx = jnp.where(inr, dst16 - rbase, 0)
                    exs = []
                    for h, amq in ((0, amax_q0), (1, amax_q1)):
                        al = alphabuf[h, sl]
                        am = plsc.load_gather(amq, [lidx])
                        ex = jnp.exp(al - am)
                        exs.append(jnp.where(inr, ex, 0.0))
                    plsc.store_compressed(lidx_c.at[pl.ds(cur, 16)], lidx,
                                          mask=inr)
                    plsc.store_compressed(src_c.at[pl.ds(cur, 16)], src16,
                                          mask=inr)
                    plsc.store_compressed(ex0_c.at[pl.ds(cur, 16)], exs[0],
                                          mask=inr)
                    plsc.store_compressed(ex1_c.at[pl.ds(cur, 16)], exs[1],
                                          mask=inr)
                    cnt = plsc.all_reduce_population_count(inr)[0]
                    cur = jnp.minimum(cur + cnt, EMAX)
                return cur

            total = lax.fori_loop(0, ITERS, chunk_scan, 0)

            def chunk_proc(z, c2):
                off = z * CH2
                pltpu.async_copy(v_hbm.at[src_c.at[pl.ds(off, CH2)]],
                                 vrows, sem).wait()

                def edge_body(e, c3):
                    ex0 = ex0_c[pl.ds(off + e, 16)][0]
                    ex1 = ex1_c[pl.ds(off + e, 16)][0]
                    dloc = lidx_c[pl.ds(off + e, 16)][0]
                    for jj in range(C // 16):
                        o1 = jj * 16
                        msgbuf[e, pl.ds(o1, 16)] = (
                            vrows[e, pl.ds(o1, 16)] * ex0)
                        o2 = C + jj * 16
                        msgbuf[e, pl.ds(o2, 16)] = (
                            vrows[e, pl.ds(o2, 16)] * ex1)
                    for tab, exh in ((dtab0, ex0), (dtab1, ex1)):
                        cur = tab[pl.ds(dloc, 16)][0]
                        plsc.store_scatter(
                            tab, [jnp.full((16,), dloc, jnp.int32)],
                            jnp.full((16,), cur + exh, jnp.float32),
                            mask=lane0)
                    return c3

                lax.fori_loop(0, CH2, edge_body, 0)
                pltpu.sync_copy(msgbuf,
                                acc_sh.at[lidx_c.at[pl.ds(off, CH2)]],
                                add=True)
                return c2

            nch = (total + CH2 - 1) // CH2
            lax.fori_loop(0, nch, chunk_proc, 0)
            plsc.subcore_barrier()

            def flush_body(z, c2):
                off = sid * RPT + z * ZCH
                pltpu.sync_copy(acc_sh.at[pl.ds(off, ZCH)],
                                msgbuf.at[pl.ds(0, ZCH)])
                pltpu.sync_copy(msgbuf.at[pl.ds(0, ZCH)],
                                acc_out.at[cid, pl.ds(rbase + off, ZCH)])
                return c2

            lax.fori_loop(0, ZITERS, flush_body, 0)
            pltpu.sync_copy(dtab0, dtab_out.at[wid, 0, pl.ds(rbase, RNG)])
            pltpu.sync_copy(dtab1, dtab_out.at[wid, 1, pl.ds(rbase, RNG)])
            plsc.subcore_barrier()
            return carry

        lax.fori_loop(0, R, range_body, 0)

    return conv_c


def _make_pool(D=128, CHP=56):
    RPT_N = N_PAD // NW
    NCH = RPT_N // CHP
    mesh = plsc.VectorSubcoreMesh(core_axis_name="c", subcore_axis_name="s",
                                  num_cores=NSC, num_subcores=TPS)

    @functools.partial(
        pl.kernel,
        out_type=[jax.ShapeDtypeStruct((NW, G, D), jnp.float32)],
        mesh=mesh,
        compiler_params=pltpu.CompilerParams(needs_layout_passes=False),
        scratch_types=[
            pltpu.VMEM((CHP, D), jnp.float32),
            pltpu.VMEM((CHP + 16,), jnp.int32),
            pltpu.VMEM((G, D), jnp.float32),
        ],
    )
    def pool(h_hbm, batch_hbm, tabs_out, rowbuf, bbuf, table):
        cid = lax.axis_index("c")
        sid = lax.axis_index("s")
        wid = sid * NSC + cid
        zv = jnp.zeros((16,), jnp.float32)

        def init_body(g, carry):
            for jj in range(D // 16):
                table[g, pl.ds(jj * 16, 16)] = zv
            return carry

        lax.fori_loop(0, G, init_body, 0)

        base0 = wid * RPT_N

        def chunk_body(it, carry):
            base = base0 + it * CHP
            pltpu.sync_copy(h_hbm.at[pl.ds(base, CHP)], rowbuf)
            pltpu.sync_copy(batch_hbm.at[pl.ds(base, CHP)],
                            bbuf.at[pl.ds(0, CHP)])

            def row_body(i, c2):
                g = bbuf[pl.ds(i, 16)][0]
                for jj in range(D // 16):
                    sl = pl.ds(jj * 16, 16)
                    table[g, sl] = jnp.maximum(table[g, sl], rowbuf[i, sl])
                return c2

            lax.fori_loop(0, CHP, row_body, 0)
            return carry

        lax.fori_loop(0, NCH, chunk_body, 0)
        pltpu.sync_copy(table, tabs_out.at[wid])

    return pool


_conv_a1 = _make_conv_a(D=64, C=32, CH=64)
_conv_a2 = _make_conv_a(D=128, C=64, CH=64)
_conv_c1 = _make_conv_c(D=64, C=32, CH=64, CH2=64, R=8, RNG=6272, EMAX=5120)
_conv_c2 = _make_conv_c(D=128, C=64, CH=64, CH2=64, R=8, RNG=6272, EMAX=5120)
_pool = _make_pool()


def kernel(x, edge_index, batch, wq1, bq1, wk1, bk1, wv1, bv1, ws1, bs1,
           wq2, bq2, wk2, bk2, wv2, bv2, ws2, bs2,
           w_lin1, b_lin1, w_lin2, b_lin2, w_lin3, b_lin3):
    src = jnp.pad(edge_index[0], (0, E_PAD - E))
    dst = jnp.pad(edge_index[1], (0, E_PAD - E))

    x8 = jnp.pad(x, ((0, 0), (0, 5)))

    def p1(w):
        return jnp.pad(w, ((0, 5), (0, 0)))

    def p128(w, b):
        return jnp.pad(w, ((0, 0), (0, 64))), jnp.pad(b, (0, 64))

    q1 = _matmul(x8, *p128(p1(wq1), bq1))
    k1 = _matmul(x8, *p128(p1(wk1), bk1))
    v1 = _matmul(x8, *p128(p1(wv1), bv1))
    s1 = _matmul(x8, p1(ws1), bs1)

    alpha1, tabs1 = _conv_a1(q1, k1, src, dst)
    amax1 = _merge_max(tabs1).reshape(2 * N_PAD)
    acc1, dtabs1 = _conv_c1(v1, src, dst, alpha1, amax1)
    denom1 = jnp.pad(_merge_sum(dtabs1).T, ((0, 0), (0, 6)))
    h1 = _norm(acc1, denom1, s1, C=32)

    q2 = _matmul(h1, wq2, bq2)
    k2 = _matmul(h1, wk2, bk2)
    v2 = _matmul(h1, wv2, bv2)
    s2 = _matmul(h1, ws2, bs2)

    alpha2, tabs2 = _conv_a2(q2, k2, src, dst)
    amax2 = _merge_max(tabs2).reshape(2 * N_PAD)
    acc2, dtabs2 = _conv_c2(v2, src, dst, alpha2, amax2)
    denom2 = jnp.pad(_merge_sum(dtabs2).T, ((0, 0), (0, 6)))
    h2 = _norm(acc2, denom2, s2, C=64)

    h2p = jnp.pad(h2, ((0, N_PAD - N), (0, 0)))
    bp = jnp.pad(batch, (0, N_PAD - N))
    (pool_tabs,) = _pool(h2p, bp)
    logits, x_latent = _mlp(pool_tabs, w_lin1, b_lin1, w_lin2, b_lin2,
                            w_lin3, b_lin3)
    return (logits, x_latent)


# speedup vs baseline: 12.9541x; 1.6315x over previous
import functools
import jax
import jax.numpy as jnp
import numpy as np
from jax import lax
from jax.experimental import pallas as pl
from jax.experimental.pallas import tpu as pltpu
from jax.experimental.pallas import tpu_sc as plsc

N = 50000
E = 800000
G = 64
NSC = 2
TPS = 16
NW = NSC * TPS
EPT = 25088
E_PAD = EPT * NW
N_PAD = 50176
NEG = -1e30
EB = 64


def _mm_body(x_ref, w_ref, b_ref, o_ref):
    o_ref[...] = (
        jnp.dot(x_ref[...], w_ref[...], preferred_element_type=jnp.float32)
        + b_ref[...]
    )


def _matmul(x, w, b, blk=1000):
    n, din = x.shape
    dout = w.shape[1]
    return pl.pallas_call(
        _mm_body,
        grid=(n // blk,),
        in_specs=[
            pl.BlockSpec((blk, din), lambda i: (i, 0)),
            pl.BlockSpec((din, dout), lambda i: (0, 0)),
            pl.BlockSpec((1, dout), lambda i: (0, 0)),
        ],
        out_specs=pl.BlockSpec((blk, dout), lambda i: (i, 0)),
        out_shape=jax.ShapeDtypeStruct((n, dout), jnp.float32),
    )(x, w, b.reshape(1, dout))


def _merge_max_body(t_ref, o_ref):
    m = jnp.max(t_ref[...], axis=0)
    o_ref[...] = jnp.where(m < -1e29, 0.0, m)


def _merge_max(tabs, blk=3584):
    nt, two, m = tabs.shape
    return pl.pallas_call(
        _merge_max_body,
        grid=(m // blk,),
        in_specs=[pl.BlockSpec((nt, two, blk), lambda i: (0, 0, i))],
        out_specs=pl.BlockSpec((two, blk), lambda i: (0, i)),
        out_shape=jax.ShapeDtypeStruct((two, m), jnp.float32),
    )(tabs)


def _merge_sum_body(t_ref, o_ref):
    o_ref[...] = jnp.sum(t_ref[...], axis=0)


def _merge_sum(tabs, blk=3584):
    nt, two, m = tabs.shape
    return pl.pallas_call(
        _merge_sum_body,
        grid=(m // blk,),
        in_specs=[pl.BlockSpec((nt, two, blk), lambda i: (0, 0, i))],
        out_specs=pl.BlockSpec((two, blk), lambda i: (0, i)),
        out_shape=jax.ShapeDtypeStruct((two, m), jnp.float32),
    )(tabs)


def _norm_body(acc_ref, dn_ref, skip_ref, o_ref, *, C):
    acc = acc_ref[0] + acc_ref[1]
    dn = dn_ref[...]
    blk = acc.shape[0]
    dnb = jnp.concatenate(
        [
            jnp.broadcast_to(dn[:, 0:1], (blk, C)),
            jnp.broadcast_to(dn[:, 1:2], (blk, C)),
        ],
        axis=1,
    )
    h = acc[:, :2 * C] / (dnb + 1e-16) + skip_ref[...]
    o_ref[...] = jnp.maximum(h, 0.0)


def _norm(acc_parts, denom2d, skip, C, blk=1000):
    n, d = skip.shape
    d2 = acc_parts.shape[2]
    return pl.pallas_call(
        functools.partial(_norm_body, C=C),
        grid=(n // blk,),
        in_specs=[
            pl.BlockSpec((2, blk, d2), lambda i: (0, i, 0)),
            pl.BlockSpec((blk, 8), lambda i: (i, 0)),
            pl.BlockSpec((blk, d), lambda i: (i, 0)),
        ],
        out_specs=pl.BlockSpec((blk, d), lambda i: (i, 0)),
        out_shape=jax.ShapeDtypeStruct((n, d), jnp.float32),
    )(acc_parts, denom2d, skip)


def _mlp_body(tabs_ref, w1_ref, b1_ref, w2_ref, b2_ref, w3_ref, b3_ref,
              logits_ref, xlat_ref):
    pooled = jnp.max(tabs_ref[...], axis=0)
    xl = jnp.maximum(
        jnp.dot(pooled, w1_ref[...], preferred_element_type=jnp.float32)
        + b1_ref[...], 0.0)
    h = jnp.maximum(
        jnp.dot(xl, w2_ref[...], preferred_element_type=jnp.float32)
        + b2_ref[...], 0.0)
    logits_ref[...] = (
        jnp.dot(h, w3_ref[...], preferred_element_type=jnp.float32)
        + b3_ref[...])
    xlat_ref[...] = xl


def _mlp(pool_tabs, w1, b1, w2, b2, w3, b3):
    nc = w3.shape[1]
    return pl.pallas_call(
        _mlp_body,
        out_shape=[
            jax.ShapeDtypeStruct((G, nc), jnp.float32),
            jax.ShapeDtypeStruct((G, 32), jnp.float32),
        ],
    )(pool_tabs, w1, b1.reshape(1, 32), w2, b2.reshape(1, 128),
      w3, b3.reshape(1, nc))


def _make_conv_a(D, C, CH, DV=128):
    ITERS = EPT // CH
    mesh = plsc.VectorSubcoreMesh(core_axis_name="c", subcore_axis_name="s",
                                  num_cores=NSC, num_subcores=TPS)
    inv_sqrt_c = float(1.0 / np.sqrt(C))

    @functools.partial(
        pl.kernel,
        out_type=[
            jax.ShapeDtypeStruct((2 * E_PAD,), jnp.float32),
            jax.ShapeDtypeStruct((NW, 2, N_PAD), jnp.float32),
        ],
        mesh=mesh,
        compiler_params=pltpu.CompilerParams(needs_layout_passes=False),
        scratch_types=[
            pltpu.VMEM((CH + 16,), jnp.int32),
            pltpu.VMEM((CH,), jnp.int32),
            pltpu.VMEM((CH, DV), jnp.float32),
            pltpu.VMEM((CH, DV), jnp.float32),
            pltpu.VMEM((2 * CH + 16,), jnp.float32),
            pltpu.VMEM((N_PAD,), jnp.float32),
            pltpu.VMEM((N_PAD,), jnp.float32),
            pltpu.SemaphoreType.DMA,
        ],
    )
    def conv_a(q_hbm, k_hbm, eb_hbm, alpha_out, tabs_out,
               dstbuf, srcbuf, qrows, krows, alphabuf, tab0, tab1, sem):
        cid = lax.axis_index("c")
        sid = lax.axis_index("s")
        wid = sid * NSC + cid
        iota = lax.iota(jnp.int32, 16)
        lane0 = iota == 0
        negv = jnp.full((16,), NEG, jnp.float32)

        def init_body(i, carry):
            tab0[pl.ds(i * 16, 16)] = negv
            tab1[pl.ds(i * 16, 16)] = negv
            return carry

        lax.fori_loop(0, N_PAD // 16, init_body, 0)

        base0 = wid * EPT

        def chunk_body(it, carry):
            base = base0 + it * CH
            blk = base // EB
            off = base - blk * EB
            pltpu.sync_copy(eb_hbm.at[pl.ds(blk * 2 * EB + off, CH)],
                            dstbuf.at[pl.ds(0, CH)])
            pltpu.sync_copy(eb_hbm.at[pl.ds(blk * 2 * EB + EB + off, CH)],
                            srcbuf)
            dq = pltpu.async_copy(q_hbm.at[dstbuf.at[pl.ds(0, CH)]], qrows,
                                  sem)
            dk = pltpu.async_copy(k_hbm.at[srcbuf], krows, sem)
            dq.wait()
            dk.wait()

            for j in range(CH // 16):
                sl = pl.ds(j * 16, 16)
                e16 = iota + j * 16
                for h in range(2):
                    acc = jnp.zeros((16,), jnp.float32)
                    for cc in range(C):
                        col = jnp.full((16,), h * C + cc, jnp.int32)
                        qc = plsc.load_gather(qrows, [e16, col])
                        kc = plsc.load_gather(krows, [e16, col])
                        acc = acc + qc * kc
                    alphabuf[pl.ds(h * CH + j * 16, 16)] = acc * inv_sqrt_c

            def edge_body(e, ecarry):
                d = dstbuf[pl.ds(e, 16)][0]
                valid = (base + e) < E
                for h, tab in ((0, tab0), (1, tab1)):
                    a = alphabuf[pl.ds(h * CH + e, 16)][0]
                    a_eff = jnp.where(valid, a, NEG)
                    cur = tab[pl.ds(d, 16)][0]
                    new = jnp.maximum(cur, a_eff)
                    plsc.store_scatter(tab, [jnp.full((16,), d, jnp.int32)],
                                       jnp.full((16,), new, jnp.float32),
                                       mask=lane0)
                return ecarry

            lax.fori_loop(0, CH, edge_body, 0)
            pltpu.sync_copy(alphabuf.at[pl.ds(0, 2 * CH)],
                            alpha_out.at[pl.ds(2 * base, 2 * CH)])
            return carry

        lax.fori_loop(0, ITERS, chunk_body, 0)
        pltpu.sync_copy(tab0, tabs_out.at[wid, 0])
        pltpu.sync_copy(tab1, tabs_out.at[wid, 1])

    return conv_a


def _make_conv_c(D, C, CH, CH2, R, RNG, EMAX, DV=128):
    ITERS = EPT // CH
    RPT = RNG // TPS
    ZCH = 8
    ZITERS = RPT // ZCH
    NJ = CH // 16
    D2 = 128
    mesh = plsc.VectorSubcoreMesh(core_axis_name="c", subcore_axis_name="s",
                                  num_cores=NSC, num_subcores=TPS)

    @functools.partial(
        pl.kernel,
        out_type=[
            jax.ShapeDtypeStruct((NSC, N_PAD, D2), jnp.float32),
            jax.ShapeDtypeStruct((NW, 2, N_PAD), jnp.float32),
        ],
        mesh=mesh,
        compiler_params=pltpu.CompilerParams(needs_layout_passes=False),
        scratch_types=[
            pltpu.VMEM((2 * CH,), jnp.int32),
            pltpu.VMEM((2 * CH,), jnp.float32),
            pltpu.VMEM((EMAX + 16,), jnp.int32),
            pltpu.VMEM((EMAX + 16,), jnp.int32),
            pltpu.VMEM((EMAX + 16,), jnp.float32),
            pltpu.VMEM((EMAX + 16,), jnp.float32),
            pltpu.VMEM((CH2, DV), jnp.float32),
            pltpu.VMEM((CH2, D2), jnp.float32),
            pltpu.VMEM((RNG,), jnp.float32),
            pltpu.VMEM((RNG,), jnp.float32),
            pltpu.VMEM((ZCH, D2), jnp.float32),
            pltpu.VMEM((RNG,), jnp.float32),
            pltpu.VMEM((RNG,), jnp.float32),
            pltpu.VMEM_SHARED((RNG, D2), jnp.float32),
            pltpu.SemaphoreType.DMA,
        ],
    )
    def conv_c(v_hbm, eb_hbm, alpha_hbm, amax_hbm,
               acc_out, dtab_out,
               ebuf, abuf, lidx_c, src_c, ex0_c, ex1_c,
               vrows, msgbuf, amax_q0, amax_q1, zbuf, dtab0, dtab1,
               acc_sh, sem):
        cid = lax.axis_index("c")
        sid = lax.axis_index("s")
        wid = sid * NSC + cid
        base0 = wid * EPT

        zv = jnp.zeros((16,), jnp.float32)
        ziv = jnp.zeros((16,), jnp.int32)
        iota = lax.iota(jnp.int32, 16)
        lane0 = iota == 0

        def zb_body(i, carry):
            for jj in range(D2 // 16):
                zbuf[i, pl.ds(jj * 16, 16)] = zv
            return carry

        lax.fori_loop(0, ZCH, zb_body, 0)

        def mz_body(i, carry):
            for jj in range(D2 // 16):
                msgbuf[i, pl.ds(jj * 16, 16)] = zv
            return carry

        lax.fori_loop(0, CH2, mz_body, 0)

        def range_body(r, carry):
            rbase = r * RNG
            pltpu.sync_copy(amax_hbm.at[pl.ds(rbase, RNG)], amax_q0)
            pltpu.sync_copy(amax_hbm.at[pl.ds(N_PAD + rbase, RNG)], amax_q1)

            def cz_body(i, c2):
                sl = pl.ds(i * 16, 16)
                lidx_c[sl] = ziv
                src_c[sl] = ziv
                ex0_c[sl] = zv
                ex1_c[sl] = zv
                return c2

            lax.fori_loop(0, (EMAX + 16) // 16, cz_body, 0)

            def dz_body(i, c2):
                sl = pl.ds(i * 16, 16)
                dtab0[sl] = zv
                dtab1[sl] = zv
                return c2

            lax.fori_loop(0, RNG // 16, dz_body, 0)

            def zero_sh(z, c2):
                pltpu.sync_copy(zbuf,
                                acc_sh.at[pl.ds(sid * RPT + z * ZCH, ZCH)])
                return c2

            lax.fori_loop(0, ZITERS, zero_sh, 0)
            plsc.subcore_barrier()

            def chunk_scan(it, cur):
                base = base0 + it * CH
                pltpu.sync_copy(eb_hbm.at[pl.ds(2 * base, 2 * CH)], ebuf)
                pltpu.sync_copy(alpha_hbm.at[pl.ds(2 * base, 2 * CH)], abuf)
                for j in range(NJ):
                    blkj = j // (EB // 16)
                    posj = j % (EB // 16)
                    dst16 = ebuf[pl.ds(blkj * 2 * EB + posj * 16, 16)]
                    src16 = ebuf[pl.ds(blkj * 2 * EB + EB + posj * 16, 16)]
                    eid = iota + (base + j * 16)
                    inr = ((dst16 >= rbase) & (dst16 < rbase + RNG)
                           & (eid < E))
                    lidx = jnp.where(inr, dst16 - rbase, 0)
                    exs = []
                    for h, amq in ((0, amax_q0), (1, amax_q1)):
                        al = abuf[pl.ds(blkj * 2 * EB + h * EB
                                        + posj * 16, 16)]
                        am = plsc.load_gather(amq, [lidx])
                        ex = jnp.exp(al - am)
                        exs.append(jnp.where(inr, ex, 0.0))
                    plsc.store_compressed(lidx_c.at[pl.ds(cur, 16)], lidx,
                                          mask=inr)
                    plsc.store_compressed(src_c.at[pl.ds(cur, 16)], src16,
                                          mask=inr)
                    plsc.store_compressed(ex0_c.at[pl.ds(cur, 16)], exs[0],
                                          mask=inr)
                    plsc.store_compressed(ex1_c.at[pl.ds(cur, 16)], exs[1],
                                          mask=inr)
                    cnt = plsc.all_reduce_population_count(inr)[0]
                    cur = jnp.minimum(cur + cnt, EMAX)
                return cur

            total = lax.fori_loop(0, ITERS, chunk_scan, 0)

            def chunk_proc(z, c2):
                off = z * CH2
                pltpu.async_copy(v_hbm.at[src_c.at[pl.ds(off, CH2)]],
                                 vrows, sem).wait()

                def edge_body(e, c3):
                    ex0 = ex0_c[pl.ds(off + e, 16)][0]
                    ex1 = ex1_c[pl.ds(off + e, 16)][0]
                    dloc = lidx_c[pl.ds(off + e, 16)][0]
                    for jj in range(C // 16):
                        o1 = jj * 16
                        msgbuf[e, pl.ds(o1, 16)] = (
                            vrows[e, pl.ds(o1, 16)] * ex0)
                        o2 = C + jj * 16
                        msgbuf[e, pl.ds(o2, 16)] = (
                            vrows[e, pl.ds(o2, 16)] * ex1)
                    for tab, exh in ((dtab0, ex0), (dtab1, ex1)):
                        cur = tab[pl.ds(dloc, 16)][0]
                        plsc.store_scatter(
                            tab, [jnp.full((16,), dloc, jnp.int32)],
                            jnp.full((16,), cur + exh, jnp.float32),
                            mask=lane0)
                    return c3

                lax.fori_loop(0, CH2, edge_body, 0)
                pltpu.sync_copy(msgbuf,
                                acc_sh.at[lidx_c.at[pl.ds(off, CH2)]],
                                add=True)
                return c2

            nch = (total + CH2 - 1) // CH2
            lax.fori_loop(0, nch, chunk_proc, 0)
            plsc.subcore_barrier()

            def flush_body(z, c2):
                off = sid * RPT + z * ZCH
                pltpu.sync_copy(acc_sh.at[pl.ds(off, ZCH)],
                                msgbuf.at[pl.ds(0, ZCH)])
                pltpu.sync_copy(msgbuf.at[pl.ds(0, ZCH)],
                                acc_out.at[cid, pl.ds(rbase + off, ZCH)])
                return c2

            lax.fori_loop(0, ZITERS, flush_body, 0)
            pltpu.sync_copy(dtab0, dtab_out.at[wid, 0, pl.ds(rbase, RNG)])
            pltpu.sync_copy(dtab1, dtab_out.at[wid, 1, pl.ds(rbase, RNG)])
            plsc.subcore_barrier()
            return carry

        lax.fori_loop(0, R, range_body, 0)

    return conv_c


def _make_pool(D=128, CHP=56):
    RPT_N = N_PAD // NW
    NCH = RPT_N // CHP
    mesh = plsc.VectorSubcoreMesh(core_axis_name="c", subcore_axis_name="s",
                                  num_cores=NSC, num_subcores=TPS)

    @functools.partial(
        pl.kernel,
        out_type=[jax.ShapeDtypeStruct((NW, G, D), jnp.float32)],
        mesh=mesh,
        compiler_params=pltpu.CompilerParams(needs_layout_passes=False),
        scratch_types=[
            pltpu.VMEM((CHP, D), jnp.float32),
            pltpu.VMEM((CHP + 16,), jnp.int32),
            pltpu.VMEM((G, D), jnp.float32),
        ],
    )
    def pool(h_hbm, batch_hbm, tabs_out, rowbuf, bbuf, table):
        cid = lax.axis_index("c")
        sid = lax.axis_index("s")
        wid = sid * NSC + cid
        zv = jnp.zeros((16,), jnp.float32)

        def init_body(g, carry):
            for jj in range(D // 16):
                table[g, pl.ds(jj * 16, 16)] = zv
            return carry

        lax.fori_loop(0, G, init_body, 0)

        base0 = wid * RPT_N

        def chunk_body(it, carry):
            base = base0 + it * CHP
            pltpu.sync_copy(h_hbm.at[pl.ds(base, CHP)], rowbuf)
            pltpu.sync_copy(batch_hbm.at[pl.ds(base, CHP)],
                            bbuf.at[pl.ds(0, CHP)])

            def row_body(i, c2):
                g = bbuf[pl.ds(i, 16)][0]
                for jj in range(D // 16):
                    sl = pl.ds(jj * 16, 16)
                    table[g, sl] = jnp.maximum(table[g, sl], rowbuf[i, sl])
                return c2

            lax.fori_loop(0, CHP, row_body, 0)
            return carry

        lax.fori_loop(0, NCH, chunk_body, 0)
        pltpu.sync_copy(table, tabs_out.at[wid])

    return pool


_conv_a1 = _make_conv_a(D=64, C=32, CH=64)
_conv_a2 = _make_conv_a(D=128, C=64, CH=64)
_conv_c1 = _make_conv_c(D=64, C=32, CH=512, CH2=64, R=8, RNG=6272, EMAX=5120)
_conv_c2 = _make_conv_c(D=128, C=64, CH=512, CH2=64, R=8, RNG=6272, EMAX=5120)
_pool = _make_pool()


def kernel(x, edge_index, batch, wq1, bq1, wk1, bk1, wv1, bv1, ws1, bs1,
           wq2, bq2, wk2, bk2, wv2, bv2, ws2, bs2,
           w_lin1, b_lin1, w_lin2, b_lin2, w_lin3, b_lin3):
    src = jnp.pad(edge_index[0], (0, E_PAD - E))
    dst = jnp.pad(edge_index[1], (0, E_PAD - E))
    eb = jnp.stack([dst.reshape(-1, EB), src.reshape(-1, EB)],
                   axis=1).reshape(-1)

    x8 = jnp.pad(x, ((0, 0), (0, 5)))

    def p1(w):
        return jnp.pad(w, ((0, 5), (0, 0)))

    def p128(w, b):
        return jnp.pad(w, ((0, 0), (0, 64))), jnp.pad(b, (0, 64))

    q1 = _matmul(x8, *p128(p1(wq1), bq1))
    k1 = _matmul(x8, *p128(p1(wk1), bk1))
    v1 = _matmul(x8, *p128(p1(wv1), bv1))
    s1 = _matmul(x8, p1(ws1), bs1)

    alpha1, tabs1 = _conv_a1(q1, k1, eb)
    amax1 = _merge_max(tabs1).reshape(2 * N_PAD)
    acc1, dtabs1 = _conv_c1(v1, eb, alpha1, amax1)
    denom1 = jnp.pad(_merge_sum(dtabs1).T, ((0, 0), (0, 6)))
    h1 = _norm(acc1, denom1, s1, C=32)

    q2 = _matmul(h1, wq2, bq2)
    k2 = _matmul(h1, wk2, bk2)
    v2 = _matmul(h1, wv2, bv2)
    s2 = _matmul(h1, ws2, bs2)

    alpha2, tabs2 = _conv_a2(q2, k2, eb)
    amax2 = _merge_max(tabs2).reshape(2 * N_PAD)
    acc2, dtabs2 = _conv_c2(v2, eb, alpha2, amax2)
    denom2 = jnp.pad(_merge_sum(dtabs2).T, ((0, 0), (0, 6)))
    h2 = _norm(acc2, denom2, s2, C=64)

    h2p = jnp.pad(h2, ((0, N_PAD - N), (0, 0)))
    bp = jnp.pad(batch, (0, N_PAD - N))
    (pool_tabs,) = _pool(h2p, bp)
    logits, x_latent = _mlp(pool_tabs, w_lin1, b_lin1, w_lin2, b_lin2,
                            w_lin3, b_lin3)
    return (logits, x_latent)
